# Initial kernel scaffold; baseline (speedup 1.0000x reference)
import jax
import jax.numpy as jnp
from jax import lax
from jax.experimental import pallas as pl
from jax.experimental.pallas import tpu as pltpu
from jax.experimental.pallas import tpu_sc as plsc

N = 10000
E = 320000
NFEAT = 128
C = 64
ALPHA = 0.2

_NC, _NS, _L = 2, 16, 16
_NW = _NC * _NS
_EW = E // _NW
_K = 80
_NCH = _EW // _K
_RPT = N // _NS

_BLK = 1000
_GRID = N // _BLK


def _mm_body(x_ref, w_ref, a2_ref, h_ref, s_ref):
    h = jnp.dot(x_ref[...], w_ref[...], preferred_element_type=jnp.float32)
    h_ref[...] = h
    s_ref[...] = jnp.dot(h, a2_ref[...], preferred_element_type=jnp.float32)


def _fin_body(acc_ref, den_ref, o_ref):
    hp = acc_ref[0] + acc_ref[1]
    d = jnp.sum(den_ref[...], axis=0)
    hn = hp / d[:, None]
    e = jnp.where(hn > 0, hn, jnp.expm1(hn))
    m = jnp.max(e, axis=1, keepdims=True)
    sh = e - m
    o_ref[...] = sh - jnp.log(jnp.sum(jnp.exp(sh), axis=1, keepdims=True))


def _sc_gat(h_hbm, s1_hbm, s2_hbm, row_hbm, col_hbm, z_hbm,
            acc_out, den_out,
            s1_v, s2_v, den_v, ridx_v, cidx_v, rows_v, w_v, acc_sh, gsem):
    cid = lax.axis_index("c")
    sid = lax.axis_index("s")
    wid = cid * _NS + sid

    pltpu.sync_copy(s1_hbm, s1_v)
    pltpu.sync_copy(s2_hbm, s2_v)

    zero16 = jnp.zeros((_L,), jnp.float32)

    def _z(i, carry):
        den_v[pl.ds(i * _L, _L)] = zero16
        return carry

    lax.fori_loop(0, N // _L, _z, 0)

    pltpu.sync_copy(z_hbm.at[pl.ds(sid * _RPT, _RPT)],
                    acc_sh.at[pl.ds(sid * _RPT, _RPT)])
    plsc.subcore_barrier()

    ebase = wid * _EW

    def _chunk(i, carry):
        base = pl.multiple_of(ebase + i * _K, _K)
        pltpu.sync_copy(row_hbm.at[pl.ds(base, _K)], ridx_v)
        pltpu.sync_copy(col_hbm.at[pl.ds(base, _K)], cidx_v)
        pltpu.async_copy(h_hbm.at[cidx_v], rows_v, gsem).wait()
        for g in range(_K // _L):
            r16 = ridx_v[pl.ds(g * _L, _L)]
            c16 = cidx_v[pl.ds(g * _L, _L)]
            z = plsc.load_gather(s1_v, [r16]) + plsc.load_gather(s2_v, [c16])
            w16 = jnp.exp(jnp.minimum(-z, -ALPHA * z))
            w_v[pl.ds(g * _L, _L)] = w16
            plsc.addupdate_scatter(den_v, [r16], w16)

        def _scale(k, carry2):
            wk = plsc.load_gather(w_v, [jnp.full((_L,), k, jnp.int32)])
            rk = rows_v.at[k]
            for q in range(C // _L):
                rk[pl.ds(q * _L, _L)] = rk[pl.ds(q * _L, _L)] * wk
            return carry2

        lax.fori_loop(0, _K, _scale, 0)
        pltpu.sync_copy(rows_v, acc_sh.at[ridx_v], add=True)
        return carry

    lax.fori_loop(0, _NCH, _chunk, 0)
    plsc.subcore_barrier()

    pltpu.sync_copy(acc_sh.at[pl.ds(sid * _RPT, _RPT)],
                    acc_out.at[cid, pl.ds(sid * _RPT, _RPT)])
    pltpu.sync_copy(den_v, den_out.at[wid])


def kernel(x, adj, W, a):
    row = adj[0].astype(jnp.int32)
    col = adj[1].astype(jnp.int32)
    a2col = jnp.stack([a[0, :C], a[0, C:]], axis=1)

    h, s12 = pl.pallas_call(
        _mm_body,
        grid=(_GRID,),
        in_specs=[
            pl.BlockSpec((_BLK, NFEAT), lambda i: (i, 0)),
            pl.BlockSpec((NFEAT, C), lambda i: (0, 0)),
            pl.BlockSpec((C, 2), lambda i: (0, 0)),
        ],
        out_specs=[
            pl.BlockSpec((_BLK, C), lambda i: (i, 0)),
            pl.BlockSpec((_BLK, 2), lambda i: (i, 0)),
        ],
        out_shape=[
            jax.ShapeDtypeStruct((N, C), jnp.float32),
            jax.ShapeDtypeStruct((N, 2), jnp.float32),
        ],
    )(x, W, a2col)

    s1 = s12[:, 0]
    s2 = s12[:, 1]
    z = jnp.zeros((N, C), jnp.float32)

    mesh = plsc.VectorSubcoreMesh(core_axis_name="c", subcore_axis_name="s",
                                  num_cores=_NC, num_subcores=_NS)
    acc, den = pl.kernel(
        _sc_gat,
        out_type=(
            jax.ShapeDtypeStruct((_NC, N, C), jnp.float32),
            jax.ShapeDtypeStruct((_NW, N), jnp.float32),
        ),
        mesh=mesh,
        scratch_types=(
            pltpu.VMEM((N,), jnp.float32),
            pltpu.VMEM((N,), jnp.float32),
            pltpu.VMEM((N,), jnp.float32),
            pltpu.VMEM((_K,), jnp.int32),
            pltpu.VMEM((_K,), jnp.int32),
            pltpu.VMEM((_K, C), jnp.float32),
            pltpu.VMEM((_K,), jnp.float32),
            pltpu.VMEM_SHARED((N, C), jnp.float32),
            pltpu.SemaphoreType.DMA,
        ),
    )(h, s1, s2, row, col, z)

    out = pl.pallas_call(
        _fin_body,
        grid=(_GRID,),
        in_specs=[
            pl.BlockSpec((_NC, _BLK, C), lambda i: (0, i, 0)),
            pl.BlockSpec((_NW, _BLK), lambda i: (0, i)),
        ],
        out_specs=pl.BlockSpec((_BLK, C), lambda i: (i, 0)),
        out_shape=jax.ShapeDtypeStruct((N, C), jnp.float32),
    )(acc, den)
    return out


# speedup vs baseline: 6.6336x; 6.6336x over previous
import jax
import jax.numpy as jnp
from jax import lax
from jax.experimental import pallas as pl
from jax.experimental.pallas import tpu as pltpu
from jax.experimental.pallas import tpu_sc as plsc

N = 10000
E = 320000
NFEAT = 128
C = 64
ALPHA = 0.2

_NC, _NS, _L = 2, 16, 16
_NW = _NC * _NS
_EW = E // _NW
_K = 80
_NCH = _EW // _K
_ROWS = [(t * 624, 624) for t in range(15)] + [(9360, 640)]

_BLK = 1000
_GRID = N // _BLK


def _mm_body(x_ref, w_ref, a2_ref, h_ref, s_ref):
    h = jnp.dot(x_ref[...], w_ref[...], preferred_element_type=jnp.float32)
    h_ref[...] = h
    s_ref[...] = jnp.dot(h, a2_ref[...], preferred_element_type=jnp.float32)


def _fin_body(acc_ref, den_ref, o_ref):
    hp = acc_ref[0] + acc_ref[1]
    d = jnp.sum(den_ref[...], axis=1)
    hn = hp / d[:, None]
    e = jnp.where(hn > 0, hn, jnp.exp(jnp.minimum(hn, 0.0)) - 1.0)
    m = jnp.max(e, axis=1, keepdims=True)
    sh = e - m
    o_ref[...] = sh - jnp.log(jnp.sum(jnp.exp(sh), axis=1, keepdims=True))


def _sc_gat(h_hbm, s1_hbm, s2_hbm, row_hbm, col_hbm, z_hbm,
            acc_out, den_out,
            s1_v, s2_v, den_v, ridx_v, cidx_v, rows_v, w_v, acc_sh, gsem):
    cid = lax.axis_index("c")
    sid = lax.axis_index("s")
    wid = cid * _NS + sid

    pltpu.sync_copy(s1_hbm, s1_v)
    pltpu.sync_copy(s2_hbm, s2_v)

    zero16 = jnp.zeros((_L,), jnp.float32)

    def _z(i, carry):
        den_v[pl.ds(i * _L, _L)] = zero16
        return carry

    lax.fori_loop(0, N // _L, _z, 0)

    for t, (off, sz) in enumerate(_ROWS):
        @pl.when(sid == t)
        def _init(off=off, sz=sz):
            pltpu.sync_copy(z_hbm.at[pl.ds(off, sz)],
                            acc_sh.at[pl.ds(off, sz)])
    plsc.subcore_barrier()

    ebase = wid * _EW

    def _chunk(i, carry):
        base = pl.multiple_of(ebase + i * _K, _K)
        pltpu.sync_copy(row_hbm.at[pl.ds(base, _K)], ridx_v)
        pltpu.sync_copy(col_hbm.at[pl.ds(base, _K)], cidx_v)
        pltpu.async_copy(h_hbm.at[cidx_v], rows_v, gsem).wait()
        for g in range(_K // _L):
            r16 = ridx_v[pl.ds(g * _L, _L)]
            c16 = cidx_v[pl.ds(g * _L, _L)]
            z = plsc.load_gather(s1_v, [r16]) + plsc.load_gather(s2_v, [c16])
            w16 = jnp.exp(jnp.minimum(-z, -ALPHA * z))
            w_v[pl.ds(g * _L, _L)] = w16
            plsc.addupdate_scatter(den_v, [r16], w16)

        def _scale(k, carry2):
            wk = plsc.load_gather(w_v, [jnp.full((_L,), k, jnp.int32)])
            rk = rows_v.at[k]
            for q in range(C // _L):
                rk[pl.ds(q * _L, _L)] = rk[pl.ds(q * _L, _L)] * wk
            return carry2

        lax.fori_loop(0, _K, _scale, 0)
        pltpu.sync_copy(rows_v, acc_sh.at[ridx_v], add=True)
        return carry

    lax.fori_loop(0, _NCH, _chunk, 0)
    plsc.subcore_barrier()

    for t, (off, sz) in enumerate(_ROWS):
        @pl.when(sid == t)
        def _wb(off=off, sz=sz):
            pltpu.sync_copy(acc_sh.at[pl.ds(off, sz)],
                            acc_out.at[cid, pl.ds(off, sz)])
    pltpu.sync_copy(den_v, den_out.at[wid, 0])


def kernel(x, adj, W, a):
    row = adj[0].astype(jnp.int32)
    col = adj[1].astype(jnp.int32)
    a2col = jnp.stack([a[0, :C], a[0, C:]], axis=1)

    h, s12 = pl.pallas_call(
        _mm_body,
        grid=(_GRID,),
        in_specs=[
            pl.BlockSpec((_BLK, NFEAT), lambda i: (i, 0)),
            pl.BlockSpec((NFEAT, C), lambda i: (0, 0)),
            pl.BlockSpec((C, 2), lambda i: (0, 0)),
        ],
        out_specs=[
            pl.BlockSpec((_BLK, C), lambda i: (i, 0)),
            pl.BlockSpec((_BLK, 2), lambda i: (i, 0)),
        ],
        out_shape=[
            jax.ShapeDtypeStruct((N, C), jnp.float32),
            jax.ShapeDtypeStruct((N, 2), jnp.float32),
        ],
    )(x, W, a2col)

    s1 = s12[:, 0]
    s2 = s12[:, 1]
    z = jnp.zeros((N, C), jnp.float32)

    mesh = plsc.VectorSubcoreMesh(core_axis_name="c", subcore_axis_name="s",
                                  num_cores=_NC, num_subcores=_NS)
    acc, den = pl.kernel(
        _sc_gat,
        out_type=(
            jax.ShapeDtypeStruct((_NC, N, C), jnp.float32),
            jax.ShapeDtypeStruct((_NW, 1, N), jnp.float32),
        ),
        mesh=mesh,
        compiler_params=pltpu.CompilerParams(use_tc_tiling_on_sc=False,
                                             needs_layout_passes=False),
        scratch_types=(
            pltpu.VMEM((N,), jnp.float32),
            pltpu.VMEM((N,), jnp.float32),
            pltpu.VMEM((N,), jnp.float32),
            pltpu.VMEM((_K,), jnp.int32),
            pltpu.VMEM((_K,), jnp.int32),
            pltpu.VMEM((_K, C), jnp.float32),
            pltpu.VMEM((_K,), jnp.float32),
            pltpu.VMEM_SHARED((N, C), jnp.float32),
            pltpu.SemaphoreType.DMA,
        ),
    )(h, s1, s2, row, col, z)

    out = pl.pallas_call(
        _fin_body,
        grid=(_GRID,),
        in_specs=[
            pl.BlockSpec((_NC, _BLK, C), lambda i: (0, i, 0)),
            pl.BlockSpec((_BLK, _NW), lambda i: (i, 0)),
        ],
        out_specs=pl.BlockSpec((_BLK, C), lambda i: (i, 0)),
        out_shape=jax.ShapeDtypeStruct((N, C), jnp.float32),
    )(acc, den.reshape(_NW, N).T)
    return out
